# Initial kernel scaffold; baseline (speedup 1.0000x reference)
import jax
import jax.numpy as jnp
from jax.experimental import pallas as pl


def kernel(x, edge_index, W1_l, W1_r, b1, W2_l, W2_r, b2):
    raise NotImplementedError("write your pallas kernel here")


# speedup vs baseline: 5.3390x; 5.3390x over previous
import jax
import jax.numpy as jnp
from jax import lax
from jax.experimental import pallas as pl
from jax.experimental.pallas import tpu as pltpu
from jax.experimental.pallas import tpu_sc as plsc

N = 10000
E = 320000
D = 128
NC = 2
NS = 16
NW = NC * NS
CHUNK = 128
NCHUNKS = E // CHUNK
BASE_CPW = NCHUNKS // NW
EXTRA = NCHUNKS - BASE_CPW * NW
NP_ = 10240
RPT = NP_ // NS
WB = 128
DEGW = 128

_MESH = plsc.VectorSubcoreMesh(
    core_axis_name="c", subcore_axis_name="s", num_cores=NC, num_subcores=NS
)


def _worker(c, s):
  wid = c * NS + s
  return wid, wid * BASE_CPW, s * RPT


@pl.kernel(
    out_type=jax.ShapeDtypeStruct((NC, NP_, D), jnp.float32),
    mesh=_MESH,
    scratch_types=[
        pltpu.VMEM((CHUNK,), jnp.int32),
        pltpu.VMEM((CHUNK,), jnp.int32),
        pltpu.VMEM((CHUNK, D), jnp.float32),
        pltpu.VMEM_SHARED((NP_, D), jnp.float32),
        pltpu.SemaphoreType.DMA,
    ],
)
def _agg(x_hbm, src_hbm, dst_hbm, zrow_hbm, out_hbm,
         src_c, dst_c, rows_v, acc_sh, sem):
  c = lax.axis_index("c")
  s = lax.axis_index("s")
  wid, cb, r0 = _worker(c, s)

  pltpu.sync_copy(zrow_hbm, rows_v)
  for k in range(RPT // WB):
    pltpu.sync_copy(rows_v, acc_sh.at[pl.ds(r0 + k * WB, WB)])
  plsc.subcore_barrier()

  def chunk_step(chunk):
    e0 = pl.multiple_of(chunk * CHUNK, 8)
    pltpu.sync_copy(src_hbm.at[pl.ds(e0, CHUNK)], src_c)
    pltpu.sync_copy(dst_hbm.at[pl.ds(e0, CHUNK)], dst_c)
    pltpu.async_copy(x_hbm.at[src_c], rows_v, sem).wait()
    pltpu.sync_copy(rows_v, acc_sh.at[dst_c], add=True)

  def step(j, carry):
    chunk_step(cb + j)
    return carry

  lax.fori_loop(0, BASE_CPW, step, 0)

  @pl.when(wid < EXTRA)
  def _():
    chunk_step(NW * BASE_CPW + wid)

  plsc.subcore_barrier()

  for k in range(RPT // WB):
    pltpu.sync_copy(acc_sh.at[pl.ds(r0 + k * WB, WB)], rows_v)
    pltpu.sync_copy(rows_v, out_hbm.at[c, pl.ds(r0 + k * WB, WB)])


@pl.kernel(
    out_type=jax.ShapeDtypeStruct((NC, NP_, DEGW), jnp.float32),
    mesh=_MESH,
    scratch_types=[
        pltpu.VMEM((CHUNK,), jnp.int32),
        pltpu.VMEM((CHUNK, DEGW), jnp.float32),
        pltpu.VMEM_SHARED((NP_, DEGW), jnp.float32),
    ],
)
def _deg(dst_hbm, zdeg_hbm, ones_hbm, deg_out_hbm, dst_c, small_v, deg_sh):
  c = lax.axis_index("c")
  s = lax.axis_index("s")
  wid, cb, r0 = _worker(c, s)

  pltpu.sync_copy(zdeg_hbm, small_v)
  for k in range(RPT // WB):
    pltpu.sync_copy(small_v, deg_sh.at[pl.ds(r0 + k * WB, WB)])
  pltpu.sync_copy(ones_hbm, small_v)
  plsc.subcore_barrier()

  def chunk_step(chunk):
    e0 = pl.multiple_of(chunk * CHUNK, 8)
    pltpu.sync_copy(dst_hbm.at[pl.ds(e0, CHUNK)], dst_c)
    pltpu.sync_copy(small_v, deg_sh.at[dst_c], add=True)

  def step(j, carry):
    chunk_step(cb + j)
    return carry

  lax.fori_loop(0, BASE_CPW, step, 0)

  @pl.when(wid < EXTRA)
  def _():
    chunk_step(NW * BASE_CPW + wid)

  plsc.subcore_barrier()

  for k in range(RPT // WB):
    pltpu.sync_copy(deg_sh.at[pl.ds(r0 + k * WB, WB)], small_v)
    pltpu.sync_copy(small_v, deg_out_hbm.at[c, pl.ds(r0 + k * WB, WB)])


def _dense(part, degp, xin, w_l, w_r, b, do_relu):
  rows = 1000

  def body(p_ref, d_ref, x_ref, wl_ref, wr_ref, b_ref, o_ref):
    agg = p_ref[0] + p_ref[1]
    deg = d_ref[0] + d_ref[1]
    degc = jnp.max(deg, axis=1, keepdims=True)
    mean = agg / jnp.maximum(degc, 1.0)
    acc = jnp.dot(mean, wl_ref[...], preferred_element_type=jnp.float32)
    acc = acc + jnp.dot(x_ref[...], wr_ref[...], preferred_element_type=jnp.float32)
    acc = acc + b_ref[...]
    if do_relu:
      acc = jnp.maximum(acc, 0.0)
    o_ref[...] = acc

  return pl.pallas_call(
      body,
      grid=(N // rows,),
      in_specs=[
          pl.BlockSpec((NC, rows, D), lambda i: (0, i, 0)),
          pl.BlockSpec((NC, rows, DEGW), lambda i: (0, i, 0)),
          pl.BlockSpec((rows, D), lambda i: (i, 0)),
          pl.BlockSpec((D, D), lambda i: (0, 0)),
          pl.BlockSpec((D, D), lambda i: (0, 0)),
          pl.BlockSpec((1, D), lambda i: (0, 0)),
      ],
      out_specs=pl.BlockSpec((rows, D), lambda i: (i, 0)),
      out_shape=jax.ShapeDtypeStruct((N, D), jnp.float32),
  )(part, degp, xin, w_l, w_r, b.reshape(1, D))


def kernel(x, edge_index, W1_l, W1_r, b1, W2_l, W2_r, b2):
  src = edge_index[0].astype(jnp.int32)
  dst = edge_index[1].astype(jnp.int32)
  zrow = jnp.zeros((WB, D), jnp.float32)
  ones = jnp.ones((CHUNK, DEGW), jnp.float32)

  degp = _deg(dst, zrow, ones)
  part1 = _agg(x, src, dst, zrow)
  h = _dense(part1, degp, x, W1_l, W1_r, b1, True)
  part2 = _agg(h, src, dst, zrow)
  out = _dense(part2, degp, h, W2_l, W2_r, b2, False)
  return out
